# baseline (speedup 1.0000x reference)
import functools

import jax
import jax.numpy as jnp
from jax import lax
from jax.experimental import pallas as pl
from jax.experimental.pallas import tpu as pltpu
from jax.experimental.pallas import tpu_sc as plsc

NC = 2
NS = 16
NW = NC * NS
CH = 128


@functools.lru_cache(maxsize=None)
def _build(B, D):
    b_per_w = B // NW
    n_ch = b_per_w // CH
    mesh = plsc.VectorSubcoreMesh(core_axis_name="c", subcore_axis_name="s")

    @functools.partial(
        pl.kernel,
        mesh=mesh,
        out_type=(
            jax.ShapeDtypeStruct((B, D), jnp.float32),
            jax.ShapeDtypeStruct((B, D), jnp.float32),
        ),
        scratch_types=[
            pltpu.VMEM((b_per_w,), jnp.int32),
            pltpu.VMEM((b_per_w,), jnp.int32),
            pltpu.VMEM((b_per_w, D), jnp.float32),
            pltpu.VMEM((b_per_w, D), jnp.float32),
            pltpu.SemaphoreType.DMA,
            pltpu.SemaphoreType.DMA,
        ],
        compiler_params=pltpu.CompilerParams(use_tc_tiling_on_sc=False),
    )
    def k(ori_hbm, dest_hbm, table_hbm, out_o_hbm, out_d_hbm,
          idx_o, idx_d, rows_o, rows_d, sem_o, sem_d):
        wid = lax.axis_index("s") * NC + lax.axis_index("c")
        base = wid * b_per_w
        pltpu.sync_copy(ori_hbm.at[pl.ds(base, b_per_w)], idx_o)
        pltpu.sync_copy(dest_hbm.at[pl.ds(base, b_per_w)], idx_d)
        copies = []
        for j in range(n_ch):
            sl = pl.ds(j * CH, CH)
            copies.append(
                pltpu.async_copy(table_hbm.at[idx_o.at[sl]], rows_o.at[sl], sem_o))
            copies.append(
                pltpu.async_copy(table_hbm.at[idx_d.at[sl]], rows_d.at[sl], sem_d))
        for c in copies:
            c.wait()
        pltpu.sync_copy(rows_o, out_o_hbm.at[pl.ds(base, b_per_w)])
        pltpu.sync_copy(rows_d, out_d_hbm.at[pl.ds(base, b_per_w)])

    return k


def kernel(ori, dest, table):
    B = ori.shape[0]
    D = table.shape[1]
    return _build(B, D)(ori.astype(jnp.int32), dest.astype(jnp.int32), table)
